# Initial kernel scaffold; baseline (speedup 1.0000x reference)
import jax
import jax.numpy as jnp
from jax.experimental import pallas as pl


def kernel(bb0, bb1, bb2, bb3, edge0, edge1, edge2, comb_w, ego_W, ego_b, W0, b0, W1, b1, W2, b2):
    raise NotImplementedError("write your pallas kernel here")


# speedup vs baseline: 29.1583x; 29.1583x over previous
import functools

import jax
import jax.numpy as jnp
from jax import lax
from jax.experimental import pallas as pl
from jax.experimental.pallas import tpu as pltpu
from jax.experimental.pallas import tpu_sc as plsc

N = 10000
E = 320000
D_IN = 128
D_OUT = 64
N_HOP = 3

NC = 2
NS = 16
NW = NC * NS
EPW = E // NW
CHUNK = 80
NCHUNK = EPW // CHUNK
NPAD = 10240
STRIPE = NPAD // NS
ROWBLK = 400
NBLK = N // ROWBLK
WPAD = 64

_mesh = plsc.VectorSubcoreMesh(core_axis_name="c", subcore_axis_name="s")


@functools.partial(
    pl.kernel,
    out_type=jax.ShapeDtypeStruct((NC * N_HOP * NPAD,), jnp.float32),
    mesh=_mesh,
    compiler_params=pltpu.CompilerParams(use_tc_tiling_on_sc=False),
    scratch_types=[
        pltpu.VMEM((NCHUNK, CHUNK), jnp.int32),
        pltpu.VMEM((CHUNK,), jnp.float32),
        pltpu.VMEM((STRIPE,), jnp.float32),
        pltpu.VMEM_SHARED((NPAD,), jnp.float32),
        pltpu.VMEM_SHARED((NPAD,), jnp.float32),
        pltpu.VMEM_SHARED((NPAD,), jnp.float32),
    ],
)
def _deg_kernel(dst_hbm, degp_hbm, idx_v, ones_v, zeros_v, acc0, acc1, acc2):
    c = lax.axis_index("c")
    s = lax.axis_index("s")
    wid = s * NC + c
    accs = (acc0, acc1, acc2)
    for i in range(CHUNK // 16):
        ones_v[pl.ds(i * 16, 16)] = jnp.ones((16,), jnp.float32)

    def _zero_body(i, _):
        zeros_v[pl.ds(i * 16, 16)] = jnp.zeros((16,), jnp.float32)
        return _

    lax.fori_loop(0, STRIPE // 16, _zero_body, 0)
    for j in range(N_HOP):
        pltpu.sync_copy(zeros_v, accs[j].at[pl.ds(s * STRIPE, STRIPE)])
    plsc.subcore_barrier()

    for j in range(N_HOP):
        pltpu.sync_copy(dst_hbm.at[j * NW + wid], idx_v)

        def _body(ch, _):
            pltpu.sync_copy(ones_v, accs[j].at[idx_v.at[ch]], add=True)
            return _

        lax.fori_loop(0, NCHUNK, _body, 0)
    plsc.subcore_barrier()
    for j in range(N_HOP):
        pltpu.sync_copy(
            accs[j].at[pl.ds(s * STRIPE, STRIPE)],
            degp_hbm.at[pl.ds((c * N_HOP + j) * NPAD + s * STRIPE, STRIPE)])


@functools.partial(
    pl.kernel,
    out_type=jax.ShapeDtypeStruct((N_HOP * NC * NPAD, WPAD), jnp.float32),
    mesh=_mesh,
    compiler_params=pltpu.CompilerParams(use_tc_tiling_on_sc=False),
    scratch_types=[
        pltpu.VMEM((NCHUNK, CHUNK), jnp.int32),
        pltpu.VMEM((NCHUNK, CHUNK), jnp.int32),
        pltpu.VMEM((CHUNK, WPAD), jnp.float32),
        pltpu.VMEM((128, WPAD), jnp.float32),
        pltpu.VMEM_SHARED((NPAD, WPAD), jnp.float32),
        pltpu.SemaphoreType.DMA,
    ],
)
def _agg_kernel(y0_hbm, y1_hbm, y2_hbm, src_hbm, dst_hbm, aggp_hbm,
                sidx, didx, gbuf, zrow, acc, sem):
    c = lax.axis_index("c")
    s = lax.axis_index("s")
    wid = s * NC + c
    y_hbm = (y0_hbm, y1_hbm, y2_hbm)

    def _zero_body(i, _):
        for k in range(WPAD // 16):
            zrow[i, pl.ds(k * 16, 16)] = jnp.zeros((16,), jnp.float32)
        return _

    lax.fori_loop(0, 128, _zero_body, 0)

    for j in range(N_HOP):
        for t in range(STRIPE // 128):
            pltpu.sync_copy(zrow, acc.at[pl.ds(s * STRIPE + t * 128, 128)])
        pltpu.sync_copy(src_hbm.at[j * NW + wid], sidx)
        pltpu.sync_copy(dst_hbm.at[j * NW + wid], didx)
        plsc.subcore_barrier()

        def _body(ch, _):
            pltpu.async_copy(y_hbm[j].at[sidx.at[ch]], gbuf, sem).wait()
            pltpu.sync_copy(gbuf, acc.at[didx.at[ch]], add=True)
            return _

        lax.fori_loop(0, NCHUNK, _body, 0)
        plsc.subcore_barrier()
        pltpu.sync_copy(
            acc.at[pl.ds(s * STRIPE, STRIPE)],
            aggp_hbm.at[pl.ds((j * NC + c) * NPAD + s * STRIPE, STRIPE)])


def _mm_body(bb1_r, bb2_r, bb3_r, wstack_r, biascat_r, degp_r,
             h0_r, y0_r, y1_r, y2_r):
    xw = (jnp.dot(bb1_r[...], wstack_r[0], preferred_element_type=jnp.float32)
          + jnp.dot(bb2_r[...], wstack_r[1], preferred_element_type=jnp.float32)
          + jnp.dot(bb3_r[...], wstack_r[2], preferred_element_type=jnp.float32)
          + biascat_r[...])
    d = degp_r[0]
    dinv = lax.rsqrt(1.0 + d[0] + d[1])
    h0_r[...] = xw[:, :D_OUT]
    y0_r[...] = xw[:, D_OUT:2 * D_OUT] * dinv[0][:, None]
    y1_r[...] = xw[:, 2 * D_OUT:3 * D_OUT] * dinv[1][:, None]
    y2_r[...] = xw[:, 3 * D_OUT:] * dinv[2][:, None]


def _mm_call(bb1, bb2, bb3, wstack, biascat, degp):
    blk = pl.BlockSpec((ROWBLK, D_IN), lambda i: (i, 0))
    hblk = pl.BlockSpec((ROWBLK, D_OUT), lambda i: (i, 0))
    yblk = pl.BlockSpec((ROWBLK, WPAD), lambda i: (i, 0))
    full_w = pl.BlockSpec(wstack.shape, lambda i: (0, 0, 0))
    full_b = pl.BlockSpec(biascat.shape, lambda i: (0, 0))
    full_d = pl.BlockSpec((1, NC, N_HOP, ROWBLK), lambda i: (i, 0, 0, 0))
    y_sd = jax.ShapeDtypeStruct((N, WPAD), jnp.float32)
    return pl.pallas_call(
        _mm_body,
        grid=(NBLK,),
        in_specs=[blk, blk, blk, full_w, full_b, full_d],
        out_specs=[hblk, yblk, yblk, yblk],
        out_shape=[jax.ShapeDtypeStruct((N, D_OUT), jnp.float32),
                   y_sd, y_sd, y_sd],
    )(bb1, bb2, bb3, wstack, biascat, degp)


def _fin_body(h0_r, y0_r, y1_r, y2_r, aggp_r, bias_r, degp_r, out_r):
    d = degp_r[0]
    dinv = lax.rsqrt(1.0 + d[0] + d[1])
    h = h0_r[...]
    ys = (y0_r, y1_r, y2_r)
    for j in range(N_HOP):
        agg = (aggp_r[j, 0, :, :D_OUT] + aggp_r[j, 1, :, :D_OUT]
               + ys[j][:, :D_OUT]) if WPAD != D_OUT else (
            aggp_r[j, 0] + aggp_r[j, 1] + ys[j][...])
        out = agg * dinv[j][:, None] + bias_r[j][None, :]
        h = h + jnp.maximum(out, 0.0)
    m = jnp.max(h, axis=1, keepdims=True)
    e = jnp.exp(h - m)
    lse = jnp.log(jnp.sum(e, axis=1, keepdims=True))
    out_r[...] = h - m - lse


def _fin_call(h0, y0, y1, y2, aggp, bias_h, degp):
    oblk = pl.BlockSpec((ROWBLK, D_OUT), lambda i: (i, 0))
    yblk = pl.BlockSpec((ROWBLK, WPAD), lambda i: (i, 0))
    ablk = pl.BlockSpec((N_HOP, NC, ROWBLK, WPAD), lambda i: (0, 0, i, 0))
    full_b = pl.BlockSpec(bias_h.shape, lambda i: (0, 0))
    full_d = pl.BlockSpec((1, NC, N_HOP, ROWBLK), lambda i: (i, 0, 0, 0))
    return pl.pallas_call(
        _fin_body,
        grid=(NBLK,),
        in_specs=[oblk, yblk, yblk, yblk, ablk, full_b, full_d],
        out_specs=oblk,
        out_shape=jax.ShapeDtypeStruct((N, D_OUT), jnp.float32),
    )(h0, y0, y1, y2, aggp, bias_h, degp)


def kernel(bb0, bb1, bb2, bb3, edge0, edge1, edge2, comb_w, ego_W, ego_b,
           W0, b0, W1, b1, W2, b2):
    del bb0
    alphas = jax.nn.softmax(comb_w, axis=1)
    wall = jnp.stack([ego_W, W0, W1, W2], axis=0)
    t = alphas[:, :, None, None] * wall[:, None, :, :]
    wstack = t.transpose(1, 2, 0, 3).reshape(3, D_IN, (N_HOP + 1) * D_OUT)
    biascat = jnp.concatenate(
        [ego_b, jnp.zeros((N_HOP * D_OUT,), jnp.float32)]).reshape(1, -1)
    bias_h = jnp.stack([b0, b1, b2], axis=0)

    edges = jnp.stack([edge0, edge1, edge2], axis=0)
    src_r = edges[:, 0].reshape(N_HOP * NW, NCHUNK, CHUNK)
    dst_r = edges[:, 1].reshape(N_HOP * NW, NCHUNK, CHUNK)

    degp = _deg_kernel(dst_r).reshape(NC, N_HOP, NPAD)[:, :, :N].reshape(
        NC, N_HOP, NBLK, ROWBLK).transpose(2, 0, 1, 3)
    h0, y0, y1, y2 = _mm_call(bb1, bb2, bb3, wstack, biascat, degp)
    aggp = _agg_kernel(y0, y1, y2, src_r, dst_r).reshape(N_HOP, NC, NPAD, WPAD)
    return _fin_call(h0, y0, y1, y2, aggp, bias_h, degp)


# speedup vs baseline: 51.9585x; 1.7819x over previous
import functools

import jax
import jax.numpy as jnp
from jax import lax
from jax.experimental import pallas as pl
from jax.experimental.pallas import tpu as pltpu
from jax.experimental.pallas import tpu_sc as plsc

N = 10000
E = 320000
D_IN = 128
D_OUT = 64
N_HOP = 3

NC = 2
NS = 16
NW = NC * NS
EPW = E // NW
CHUNK = 80
NCHUNK = EPW // CHUNK
NPAD = 10240
STRIPE = NPAD // NS
ROWBLK = 400
NBLK = N // ROWBLK
WPAD = 64

_mesh = plsc.VectorSubcoreMesh(core_axis_name="c", subcore_axis_name="s")


@functools.partial(
    pl.kernel,
    out_type=jax.ShapeDtypeStruct((NC * N_HOP * NPAD,), jnp.float32),
    mesh=_mesh,
    compiler_params=pltpu.CompilerParams(use_tc_tiling_on_sc=False),
    scratch_types=[
        pltpu.VMEM((NCHUNK, CHUNK), jnp.int32),
        pltpu.VMEM((CHUNK,), jnp.float32),
        pltpu.VMEM((STRIPE,), jnp.float32),
        pltpu.VMEM_SHARED((NPAD,), jnp.float32),
        pltpu.VMEM_SHARED((NPAD,), jnp.float32),
        pltpu.VMEM_SHARED((NPAD,), jnp.float32),
    ],
)
def _deg_kernel(dst_hbm, degp_hbm, idx_v, ones_v, zeros_v, acc0, acc1, acc2):
    c = lax.axis_index("c")
    s = lax.axis_index("s")
    wid = s * NC + c
    accs = (acc0, acc1, acc2)
    for i in range(CHUNK // 16):
        ones_v[pl.ds(i * 16, 16)] = jnp.ones((16,), jnp.float32)

    def _zero_body(i, _):
        zeros_v[pl.ds(i * 16, 16)] = jnp.zeros((16,), jnp.float32)
        return _

    lax.fori_loop(0, STRIPE // 16, _zero_body, 0)
    for j in range(N_HOP):
        pltpu.sync_copy(zeros_v, accs[j].at[pl.ds(s * STRIPE, STRIPE)])
    plsc.subcore_barrier()

    for j in range(N_HOP):
        pltpu.sync_copy(dst_hbm.at[j * NW + wid], idx_v)

        def _body(ch, _):
            pltpu.sync_copy(ones_v, accs[j].at[idx_v.at[ch]], add=True)
            return _

        lax.fori_loop(0, NCHUNK, _body, 0)
    plsc.subcore_barrier()
    for j in range(N_HOP):
        pltpu.sync_copy(
            accs[j].at[pl.ds(s * STRIPE, STRIPE)],
            degp_hbm.at[pl.ds((c * N_HOP + j) * NPAD + s * STRIPE, STRIPE)])


NBUF = 5


@functools.partial(
    pl.kernel,
    out_type=jax.ShapeDtypeStruct((N_HOP * NC * NPAD, WPAD), jnp.float32),
    mesh=_mesh,
    compiler_params=pltpu.CompilerParams(use_tc_tiling_on_sc=False),
    scratch_types=[
        pltpu.VMEM((NCHUNK, CHUNK), jnp.int32),
        pltpu.VMEM((NCHUNK, CHUNK), jnp.int32),
        [pltpu.VMEM((CHUNK, WPAD), jnp.float32) for _ in range(NBUF)],
        pltpu.VMEM((128, WPAD), jnp.float32),
        pltpu.VMEM_SHARED((NPAD, WPAD), jnp.float32),
        [pltpu.SemaphoreType.DMA for _ in range(NBUF)],
        [pltpu.SemaphoreType.DMA for _ in range(NBUF)],
    ],
)
def _agg_kernel(y0_hbm, y1_hbm, y2_hbm, src_hbm, dst_hbm, aggp_hbm,
                sidx, didx, gbufs, zrow, acc, gsems, ssems):
    c = lax.axis_index("c")
    s = lax.axis_index("s")
    wid = s * NC + c
    y_hbm = (y0_hbm, y1_hbm, y2_hbm)

    def _zero_body(i, _):
        for k in range(WPAD // 16):
            zrow[i, pl.ds(k * 16, 16)] = jnp.zeros((16,), jnp.float32)
        return _

    lax.fori_loop(0, 128, _zero_body, 0)

    for j in range(N_HOP):
        for t in range(STRIPE // 128):
            pltpu.sync_copy(zrow, acc.at[pl.ds(s * STRIPE + t * 128, 128)])
        pltpu.sync_copy(src_hbm.at[j * NW + wid], sidx)
        pltpu.sync_copy(dst_hbm.at[j * NW + wid], didx)
        plsc.subcore_barrier()

        for b in range(NBUF):
            pltpu.async_copy(y_hbm[j].at[sidx.at[b]], gbufs[b], gsems[b])

        def _group(g, carry):
            for b in range(NBUF):
                ch = g * NBUF + b
                pltpu.make_async_copy(
                    y_hbm[j].at[pl.ds(0, CHUNK)], gbufs[b], gsems[b]).wait()
                pltpu.async_copy(gbufs[b], acc.at[didx.at[ch]], ssems[b],
                                 add=True)
                pltpu.make_async_copy(
                    gbufs[b], acc.at[pl.ds(0, CHUNK)], ssems[b]).wait()

                @pl.when(g < NCHUNK // NBUF - 1)
                def _prefetch():
                    pltpu.async_copy(y_hbm[j].at[sidx.at[ch + NBUF]],
                                     gbufs[b], gsems[b])
            return carry

        lax.fori_loop(0, NCHUNK // NBUF, _group, 0)
        plsc.subcore_barrier()
        pltpu.sync_copy(
            acc.at[pl.ds(s * STRIPE, STRIPE)],
            aggp_hbm.at[pl.ds((j * NC + c) * NPAD + s * STRIPE, STRIPE)])


def _mm_body(bb1_r, bb2_r, bb3_r, wstack_r, biascat_r, degp_r,
             h0_r, y0_r, y1_r, y2_r):
    xw = (jnp.dot(bb1_r[...], wstack_r[0], preferred_element_type=jnp.float32)
          + jnp.dot(bb2_r[...], wstack_r[1], preferred_element_type=jnp.float32)
          + jnp.dot(bb3_r[...], wstack_r[2], preferred_element_type=jnp.float32)
          + biascat_r[...])
    d = degp_r[0]
    dinv = lax.rsqrt(1.0 + d[0] + d[1])
    h0_r[...] = xw[:, :D_OUT]
    y0_r[...] = xw[:, D_OUT:2 * D_OUT] * dinv[0][:, None]
    y1_r[...] = xw[:, 2 * D_OUT:3 * D_OUT] * dinv[1][:, None]
    y2_r[...] = xw[:, 3 * D_OUT:] * dinv[2][:, None]


def _mm_call(bb1, bb2, bb3, wstack, biascat, degp):
    blk = pl.BlockSpec((ROWBLK, D_IN), lambda i: (i, 0))
    hblk = pl.BlockSpec((ROWBLK, D_OUT), lambda i: (i, 0))
    yblk = pl.BlockSpec((ROWBLK, WPAD), lambda i: (i, 0))
    full_w = pl.BlockSpec(wstack.shape, lambda i: (0, 0, 0))
    full_b = pl.BlockSpec(biascat.shape, lambda i: (0, 0))
    full_d = pl.BlockSpec((1, NC, N_HOP, ROWBLK), lambda i: (i, 0, 0, 0))
    y_sd = jax.ShapeDtypeStruct((N, WPAD), jnp.float32)
    return pl.pallas_call(
        _mm_body,
        grid=(NBLK,),
        in_specs=[blk, blk, blk, full_w, full_b, full_d],
        out_specs=[hblk, yblk, yblk, yblk],
        out_shape=[jax.ShapeDtypeStruct((N, D_OUT), jnp.float32),
                   y_sd, y_sd, y_sd],
    )(bb1, bb2, bb3, wstack, biascat, degp)


def _fin_body(h0_r, y0_r, y1_r, y2_r, aggp_r, bias_r, degp_r, out_r):
    d = degp_r[0]
    dinv = lax.rsqrt(1.0 + d[0] + d[1])
    h = h0_r[...]
    ys = (y0_r, y1_r, y2_r)
    for j in range(N_HOP):
        agg = (aggp_r[j, 0, :, :D_OUT] + aggp_r[j, 1, :, :D_OUT]
               + ys[j][:, :D_OUT]) if WPAD != D_OUT else (
            aggp_r[j, 0] + aggp_r[j, 1] + ys[j][...])
        out = agg * dinv[j][:, None] + bias_r[j][None, :]
        h = h + jnp.maximum(out, 0.0)
    m = jnp.max(h, axis=1, keepdims=True)
    e = jnp.exp(h - m)
    lse = jnp.log(jnp.sum(e, axis=1, keepdims=True))
    out_r[...] = h - m - lse


def _fin_call(h0, y0, y1, y2, aggp, bias_h, degp):
    oblk = pl.BlockSpec((ROWBLK, D_OUT), lambda i: (i, 0))
    yblk = pl.BlockSpec((ROWBLK, WPAD), lambda i: (i, 0))
    ablk = pl.BlockSpec((N_HOP, NC, ROWBLK, WPAD), lambda i: (0, 0, i, 0))
    full_b = pl.BlockSpec(bias_h.shape, lambda i: (0, 0))
    full_d = pl.BlockSpec((1, NC, N_HOP, ROWBLK), lambda i: (i, 0, 0, 0))
    return pl.pallas_call(
        _fin_body,
        grid=(NBLK,),
        in_specs=[oblk, yblk, yblk, yblk, ablk, full_b, full_d],
        out_specs=oblk,
        out_shape=jax.ShapeDtypeStruct((N, D_OUT), jnp.float32),
    )(h0, y0, y1, y2, aggp, bias_h, degp)


def kernel(bb0, bb1, bb2, bb3, edge0, edge1, edge2, comb_w, ego_W, ego_b,
           W0, b0, W1, b1, W2, b2):
    del bb0
    alphas = jax.nn.softmax(comb_w, axis=1)
    wall = jnp.stack([ego_W, W0, W1, W2], axis=0)
    t = alphas[:, :, None, None] * wall[:, None, :, :]
    wstack = t.transpose(1, 2, 0, 3).reshape(3, D_IN, (N_HOP + 1) * D_OUT)
    biascat = jnp.concatenate(
        [ego_b, jnp.zeros((N_HOP * D_OUT,), jnp.float32)]).reshape(1, -1)
    bias_h = jnp.stack([b0, b1, b2], axis=0)

    edges = jnp.stack([edge0, edge1, edge2], axis=0)
    src_r = edges[:, 0].reshape(N_HOP * NW, NCHUNK, CHUNK)
    dst_r = edges[:, 1].reshape(N_HOP * NW, NCHUNK, CHUNK)

    degp = _deg_kernel(dst_r).reshape(NC, N_HOP, NPAD)[:, :, :N].reshape(
        NC, N_HOP, NBLK, ROWBLK).transpose(2, 0, 1, 3)
    h0, y0, y1, y2 = _mm_call(bb1, bb2, bb3, wstack, biascat, degp)
    aggp = _agg_kernel(y0, y1, y2, src_r, dst_r).reshape(N_HOP, NC, NPAD, WPAD)
    return _fin_call(h0, y0, y1, y2, aggp, bias_h, degp)


# speedup vs baseline: 53.6410x; 1.0324x over previous
import functools

import jax
import jax.numpy as jnp
from jax import lax
from jax.experimental import pallas as pl
from jax.experimental.pallas import tpu as pltpu
from jax.experimental.pallas import tpu_sc as plsc

N = 10000
E = 320000
D_IN = 128
D_OUT = 64
N_HOP = 3

NC = 2
NS = 16
NW = NC * NS
EPW = E // NW
CHUNK = 80
NCHUNK = EPW // CHUNK
NPAD = 10240
STRIPE = NPAD // NS
WPAD = 64
RB = 512
NRB = NPAD // RB
NBUF = 5
LOOKA = 3

_mesh = plsc.VectorSubcoreMesh(core_axis_name="c", subcore_axis_name="s")


@functools.partial(
    pl.kernel,
    out_type=jax.ShapeDtypeStruct((NC * N_HOP * NPAD,), jnp.float32),
    mesh=_mesh,
    compiler_params=pltpu.CompilerParams(use_tc_tiling_on_sc=False),
    scratch_types=[
        pltpu.VMEM((NCHUNK, CHUNK), jnp.int32),
        pltpu.VMEM((CHUNK,), jnp.float32),
        pltpu.VMEM((STRIPE,), jnp.float32),
        pltpu.VMEM_SHARED((NPAD,), jnp.float32),
        pltpu.VMEM_SHARED((NPAD,), jnp.float32),
        pltpu.VMEM_SHARED((NPAD,), jnp.float32),
    ],
)
def _deg_kernel(dst0, dst1, dst2, degp_hbm, idx_v, ones_v, zeros_v,
                acc0, acc1, acc2):
    c = lax.axis_index("c")
    s = lax.axis_index("s")
    wid = s * NC + c
    dsts = (dst0, dst1, dst2)
    accs = (acc0, acc1, acc2)
    for i in range(CHUNK // 16):
        ones_v[pl.ds(i * 16, 16)] = jnp.ones((16,), jnp.float32)

    def _zero_body(i, carry):
        zeros_v[pl.ds(i * 16, 16)] = jnp.zeros((16,), jnp.float32)
        return carry

    lax.fori_loop(0, STRIPE // 16, _zero_body, 0)
    for j in range(N_HOP):
        pltpu.sync_copy(zeros_v, accs[j].at[pl.ds(s * STRIPE, STRIPE)])
    plsc.subcore_barrier()

    for j in range(N_HOP):
        pltpu.sync_copy(dsts[j].at[wid], idx_v)

        def _body(ch, carry):
            pltpu.sync_copy(ones_v, accs[j].at[idx_v.at[ch]], add=True)
            return carry

        lax.fori_loop(0, NCHUNK, _body, 0)
    plsc.subcore_barrier()
    for j in range(N_HOP):
        pltpu.sync_copy(
            accs[j].at[pl.ds(s * STRIPE, STRIPE)],
            degp_hbm.at[pl.ds((c * N_HOP + j) * NPAD + s * STRIPE, STRIPE)])


@functools.partial(
    pl.kernel,
    out_type=jax.ShapeDtypeStruct((N_HOP * NC * NPAD, WPAD), jnp.float32),
    mesh=_mesh,
    compiler_params=pltpu.CompilerParams(use_tc_tiling_on_sc=False),
    scratch_types=[
        pltpu.VMEM((NCHUNK, CHUNK), jnp.int32),
        pltpu.VMEM((NCHUNK, CHUNK), jnp.int32),
        [pltpu.VMEM((CHUNK, WPAD), jnp.float32) for _ in range(NBUF)],
        pltpu.VMEM((128, WPAD), jnp.float32),
        pltpu.VMEM_SHARED((NPAD, WPAD), jnp.float32),
        [pltpu.SemaphoreType.DMA for _ in range(NBUF)],
        [pltpu.SemaphoreType.DMA for _ in range(NBUF)],
    ],
)
def _agg_kernel(y0_hbm, y1_hbm, y2_hbm, src0, src1, src2, dst0, dst1, dst2,
                aggp_hbm, sidx, didx, gbufs, zrow, acc, gsems, ssems):
    c = lax.axis_index("c")
    s = lax.axis_index("s")
    wid = s * NC + c
    y_hbm = (y0_hbm, y1_hbm, y2_hbm)
    srcs = (src0, src1, src2)
    dsts = (dst0, dst1, dst2)

    def _zero_body(i, carry):
        for k in range(WPAD // 16):
            zrow[i, pl.ds(k * 16, 16)] = jnp.zeros((16,), jnp.float32)
        return carry

    lax.fori_loop(0, 128, _zero_body, 0)

    for j in range(N_HOP):
        for t in range(STRIPE // 128):
            pltpu.sync_copy(zrow, acc.at[pl.ds(s * STRIPE + t * 128, 128)])
        pltpu.sync_copy(srcs[j].at[wid], sidx)
        pltpu.sync_copy(dsts[j].at[wid], didx)
        plsc.subcore_barrier()

        for b in range(LOOKA):
            pltpu.async_copy(y_hbm[j].at[sidx.at[b]], gbufs[b], gsems[b])

        def _group(g, carry):
            for b in range(NBUF):
                ch = g * NBUF + b
                ch_pf = ch + LOOKA
                slot_pf = (b + LOOKA) % NBUF

                @pl.when(jnp.logical_and(ch_pf >= NBUF, ch_pf < NCHUNK))
                def _drain():
                    pltpu.make_async_copy(gbufs[slot_pf],
                                          acc.at[pl.ds(0, CHUNK)],
                                          ssems[slot_pf]).wait()

                @pl.when(ch_pf < NCHUNK)
                def _prefetch():
                    pltpu.async_copy(y_hbm[j].at[sidx.at[ch_pf]],
                                     gbufs[slot_pf], gsems[slot_pf])

                pltpu.make_async_copy(y_hbm[j].at[pl.ds(0, CHUNK)],
                                      gbufs[b], gsems[b]).wait()
                pltpu.async_copy(gbufs[b], acc.at[didx.at[ch]], ssems[b],
                                 add=True)
            return carry

        lax.fori_loop(0, NCHUNK // NBUF, _group, 0)
        for b in range(NBUF):
            pltpu.make_async_copy(gbufs[b], acc.at[pl.ds(0, CHUNK)],
                                  ssems[b]).wait()
        plsc.subcore_barrier()
        pltpu.sync_copy(
            acc.at[pl.ds(s * STRIPE, STRIPE)],
            aggp_hbm.at[pl.ds((j * NC + c) * NPAD + s * STRIPE, STRIPE)])


def _mm_body(bb1_r, bb2_r, bb3_r, wstack_r, biascat_r, degp_r,
             h0_r, y0_r, y1_r, y2_r):
    xw = (jnp.dot(bb1_r[...], wstack_r[0], preferred_element_type=jnp.float32)
          + jnp.dot(bb2_r[...], wstack_r[1], preferred_element_type=jnp.float32)
          + jnp.dot(bb3_r[...], wstack_r[2], preferred_element_type=jnp.float32)
          + biascat_r[...])
    d = degp_r[...]
    h0_r[...] = xw[:, :D_OUT]
    ys = (y0_r, y1_r, y2_r)
    for j in range(N_HOP):
        dinv = lax.rsqrt(1.0 + d[j] + d[N_HOP + j])
        ys[j][...] = xw[:, (j + 1) * D_OUT:(j + 2) * D_OUT] * dinv[:, None]


def _mm_call(bb1, bb2, bb3, wstack, biascat, degp):
    blk = pl.BlockSpec((RB, D_IN), lambda i: (i, 0))
    oblk = pl.BlockSpec((RB, D_OUT), lambda i: (i, 0))
    dblk = pl.BlockSpec((NC * N_HOP, RB), lambda i: (0, i))
    y_sd = jax.ShapeDtypeStruct((N, WPAD), jnp.float32)
    return pl.pallas_call(
        _mm_body,
        grid=(NRB,),
        in_specs=[blk, blk, blk,
                  pl.BlockSpec(wstack.shape, lambda i: (0, 0, 0)),
                  pl.BlockSpec(biascat.shape, lambda i: (0, 0)),
                  dblk],
        out_specs=[oblk, oblk, oblk, oblk],
        out_shape=[jax.ShapeDtypeStruct((N, D_OUT), jnp.float32),
                   y_sd, y_sd, y_sd],
    )(bb1, bb2, bb3, wstack, biascat, degp)


def _fin_body(h0_r, y0_r, y1_r, y2_r, aggp_r, bias_r, degp_r, out_r):
    d = degp_r[...]
    h = h0_r[...]
    ys = (y0_r, y1_r, y2_r)
    for j in range(N_HOP):
        dinv = lax.rsqrt(1.0 + d[j] + d[N_HOP + j])
        agg = aggp_r[j * NC] + aggp_r[j * NC + 1] + ys[j][...]
        out = agg * dinv[:, None] + bias_r[j][None, :]
        h = h + jnp.maximum(out, 0.0)
    m = jnp.max(h, axis=1, keepdims=True)
    e = jnp.exp(h - m)
    lse = jnp.log(jnp.sum(e, axis=1, keepdims=True))
    out_r[...] = h - m - lse


def _fin_call(h0, y0, y1, y2, aggp, bias_h, degp):
    oblk = pl.BlockSpec((RB, D_OUT), lambda i: (i, 0))
    ablk = pl.BlockSpec((N_HOP * NC, RB, WPAD), lambda i: (0, i, 0))
    dblk = pl.BlockSpec((NC * N_HOP, RB), lambda i: (0, i))
    return pl.pallas_call(
        _fin_body,
        grid=(NRB,),
        in_specs=[oblk, oblk, oblk, oblk, ablk,
                  pl.BlockSpec(bias_h.shape, lambda i: (0, 0)),
                  dblk],
        out_specs=oblk,
        out_shape=jax.ShapeDtypeStruct((N, D_OUT), jnp.float32),
    )(h0, y0, y1, y2, aggp, bias_h, degp)


def kernel(bb0, bb1, bb2, bb3, edge0, edge1, edge2, comb_w, ego_W, ego_b,
           W0, b0, W1, b1, W2, b2):
    del bb0
    alphas = jax.nn.softmax(comb_w, axis=1)
    wall = jnp.stack([ego_W, W0, W1, W2], axis=0)
    t = alphas[:, :, None, None] * wall[:, None, :, :]
    wstack = t.transpose(1, 2, 0, 3).reshape(3, D_IN, (N_HOP + 1) * D_OUT)
    biascat = jnp.concatenate(
        [ego_b, jnp.zeros((N_HOP * D_OUT,), jnp.float32)]).reshape(1, -1)
    bias_h = jnp.stack([b0, b1, b2], axis=0)

    src0 = edge0[0].reshape(NW, NCHUNK, CHUNK)
    dst0 = edge0[1].reshape(NW, NCHUNK, CHUNK)
    src1 = edge1[0].reshape(NW, NCHUNK, CHUNK)
    dst1 = edge1[1].reshape(NW, NCHUNK, CHUNK)
    src2 = edge2[0].reshape(NW, NCHUNK, CHUNK)
    dst2 = edge2[1].reshape(NW, NCHUNK, CHUNK)

    degp = _deg_kernel(dst0, dst1, dst2).reshape(NC * N_HOP, NPAD)
    h0, y0, y1, y2 = _mm_call(bb1, bb2, bb3, wstack, biascat, degp)
    aggp = _agg_kernel(y0, y1, y2, src0, src1, src2,
                       dst0, dst1, dst2).reshape(N_HOP * NC, NPAD, WPAD)
    return _fin_call(h0, y0, y1, y2, aggp, bias_h, degp)


# speedup vs baseline: 62.0649x; 1.1570x over previous
import functools

import jax
import jax.numpy as jnp
from jax import lax
from jax.experimental import pallas as pl
from jax.experimental.pallas import tpu as pltpu
from jax.experimental.pallas import tpu_sc as plsc

N = 10000
E = 320000
D_IN = 128
D_OUT = 64
N_HOP = 3

NC = 2
NS = 16
NW = NC * NS
EPW = E // NW
CHUNK = 128
NFULL = EPW // CHUNK
TAIL = EPW - NFULL * CHUNK
NPAD = 10240
STRIPE = NPAD // NS
WPAD = 64
RB = 512
NRB = NPAD // RB
NBUF = 6
LOOKA = 3

_mesh = plsc.VectorSubcoreMesh(core_axis_name="c", subcore_axis_name="s")


@functools.partial(
    pl.kernel,
    out_type=jax.ShapeDtypeStruct((NC * N_HOP * NPAD,), jnp.float32),
    mesh=_mesh,
    compiler_params=pltpu.CompilerParams(use_tc_tiling_on_sc=False),
    scratch_types=[
        pltpu.VMEM((EPW,), jnp.int32),
        pltpu.VMEM((CHUNK,), jnp.float32),
        pltpu.VMEM((STRIPE,), jnp.float32),
        pltpu.VMEM_SHARED((NPAD,), jnp.float32),
        pltpu.VMEM_SHARED((NPAD,), jnp.float32),
        pltpu.VMEM_SHARED((NPAD,), jnp.float32),
    ],
)
def _deg_kernel(edge0, edge1, edge2, degp_hbm, idx_v, ones_v, zeros_v,
                acc0, acc1, acc2):
    c = lax.axis_index("c")
    s = lax.axis_index("s")
    wid = s * NC + c
    edges = (edge0, edge1, edge2)
    accs = (acc0, acc1, acc2)
    for i in range(CHUNK // 16):
        ones_v[pl.ds(i * 16, 16)] = jnp.ones((16,), jnp.float32)

    def _zero_body(i, carry):
        zeros_v[pl.ds(i * 16, 16)] = jnp.zeros((16,), jnp.float32)
        return carry

    lax.fori_loop(0, STRIPE // 16, _zero_body, 0)
    for j in range(N_HOP):
        pltpu.sync_copy(zeros_v, accs[j].at[pl.ds(s * STRIPE, STRIPE)])
    plsc.subcore_barrier()

    for j in range(N_HOP):
        pltpu.sync_copy(edges[j].at[1, pl.ds(wid * EPW, EPW)], idx_v)

        def _body(ch, carry):
            pltpu.sync_copy(ones_v,
                            accs[j].at[idx_v.at[pl.ds(ch * CHUNK, CHUNK)]],
                            add=True)
            return carry

        lax.fori_loop(0, NFULL, _body, 0)
        pltpu.sync_copy(ones_v.at[pl.ds(0, TAIL)],
                        accs[j].at[idx_v.at[pl.ds(NFULL * CHUNK, TAIL)]],
                        add=True)
    plsc.subcore_barrier()
    for j in range(N_HOP):
        pltpu.sync_copy(
            accs[j].at[pl.ds(s * STRIPE, STRIPE)],
            degp_hbm.at[pl.ds((c * N_HOP + j) * NPAD + s * STRIPE, STRIPE)])


@functools.partial(
    pl.kernel,
    out_type=jax.ShapeDtypeStruct((N_HOP * NC * NPAD, WPAD), jnp.float32),
    mesh=_mesh,
    compiler_params=pltpu.CompilerParams(use_tc_tiling_on_sc=False),
    scratch_types=[
        pltpu.VMEM((EPW,), jnp.int32),
        pltpu.VMEM((EPW,), jnp.int32),
        [pltpu.VMEM((CHUNK, WPAD), jnp.float32) for _ in range(NBUF)],
        pltpu.VMEM((128, WPAD), jnp.float32),
        pltpu.VMEM_SHARED((NPAD, WPAD), jnp.float32),
        [pltpu.SemaphoreType.DMA for _ in range(NBUF)],
        [pltpu.SemaphoreType.DMA for _ in range(NBUF)],
    ],
)
def _agg_kernel(y0_hbm, y1_hbm, y2_hbm, edge0, edge1, edge2,
                aggp_hbm, sidx, didx, gbufs, zrow, acc, gsems, ssems):
    c = lax.axis_index("c")
    s = lax.axis_index("s")
    wid = s * NC + c
    y_hbm = (y0_hbm, y1_hbm, y2_hbm)
    edges = (edge0, edge1, edge2)

    def _zero_body(i, carry):
        for k in range(WPAD // 16):
            zrow[i, pl.ds(k * 16, 16)] = jnp.zeros((16,), jnp.float32)
        return carry

    lax.fori_loop(0, 128, _zero_body, 0)

    for j in range(N_HOP):
        for t in range(STRIPE // 128):
            pltpu.sync_copy(zrow, acc.at[pl.ds(s * STRIPE + t * 128, 128)])
        pltpu.sync_copy(edges[j].at[0, pl.ds(wid * EPW, EPW)], sidx)
        pltpu.sync_copy(edges[j].at[1, pl.ds(wid * EPW, EPW)], didx)
        plsc.subcore_barrier()

        for b in range(LOOKA):
            pltpu.async_copy(y_hbm[j].at[sidx.at[pl.ds(b * CHUNK, CHUNK)]],
                             gbufs[b], gsems[b])

        def _group(g, carry):
            for b in range(NBUF):
                ch = g * NBUF + b
                ch_pf = ch + LOOKA
                slot_pf = (b + LOOKA) % NBUF

                @pl.when(jnp.logical_and(ch_pf >= NBUF, ch_pf < NFULL))
                def _drain():
                    pltpu.make_async_copy(gbufs[slot_pf],
                                          acc.at[pl.ds(0, CHUNK)],
                                          ssems[slot_pf]).wait()

                @pl.when(ch_pf < NFULL)
                def _prefetch():
                    pltpu.async_copy(
                        y_hbm[j].at[sidx.at[pl.ds(ch_pf * CHUNK, CHUNK)]],
                        gbufs[slot_pf], gsems[slot_pf])

                pltpu.make_async_copy(y_hbm[j].at[pl.ds(0, CHUNK)],
                                      gbufs[b], gsems[b]).wait()
                pltpu.async_copy(gbufs[b],
                                 acc.at[didx.at[pl.ds(ch * CHUNK, CHUNK)]],
                                 ssems[b], add=True)
            return carry

        lax.fori_loop(0, NFULL // NBUF, _group, 0)
        for b in range(NBUF):
            pltpu.make_async_copy(gbufs[b], acc.at[pl.ds(0, CHUNK)],
                                  ssems[b]).wait()
        pltpu.async_copy(
            y_hbm[j].at[sidx.at[pl.ds(NFULL * CHUNK, TAIL)]],
            gbufs[0].at[pl.ds(0, TAIL)], gsems[0]).wait()
        pltpu.sync_copy(gbufs[0].at[pl.ds(0, TAIL)],
                        acc.at[didx.at[pl.ds(NFULL * CHUNK, TAIL)]],
                        add=True)
        plsc.subcore_barrier()
        pltpu.sync_copy(
            acc.at[pl.ds(s * STRIPE, STRIPE)],
            aggp_hbm.at[pl.ds((j * NC + c) * NPAD + s * STRIPE, STRIPE)])


def _mm_body(bb1_r, bb2_r, bb3_r, wstack_r, biascat_r, degp_r,
             h0_r, y0_r, y1_r, y2_r):
    xw = (jnp.dot(bb1_r[...], wstack_r[0], preferred_element_type=jnp.float32)
          + jnp.dot(bb2_r[...], wstack_r[1], preferred_element_type=jnp.float32)
          + jnp.dot(bb3_r[...], wstack_r[2], preferred_element_type=jnp.float32)
          + biascat_r[...])
    d = degp_r[...]
    h0_r[...] = xw[:, :D_OUT]
    ys = (y0_r, y1_r, y2_r)
    for j in range(N_HOP):
        dinv = lax.rsqrt(1.0 + d[j] + d[N_HOP + j])
        ys[j][...] = xw[:, (j + 1) * D_OUT:(j + 2) * D_OUT] * dinv[:, None]


def _mm_call(bb1, bb2, bb3, wstack, biascat, degp):
    blk = pl.BlockSpec((RB, D_IN), lambda i: (i, 0))
    oblk = pl.BlockSpec((RB, D_OUT), lambda i: (i, 0))
    dblk = pl.BlockSpec((NC * N_HOP, RB), lambda i: (0, i))
    y_sd = jax.ShapeDtypeStruct((N, WPAD), jnp.float32)
    return pl.pallas_call(
        _mm_body,
        grid=(NRB,),
        in_specs=[blk, blk, blk,
                  pl.BlockSpec(wstack.shape, lambda i: (0, 0, 0)),
                  pl.BlockSpec(biascat.shape, lambda i: (0, 0)),
                  dblk],
        out_specs=[oblk, oblk, oblk, oblk],
        out_shape=[jax.ShapeDtypeStruct((N, D_OUT), jnp.float32),
                   y_sd, y_sd, y_sd],
    )(bb1, bb2, bb3, wstack, biascat, degp)


def _fin_body(h0_r, y0_r, y1_r, y2_r, aggp_r, bias_r, degp_r, out_r):
    d = degp_r[...]
    h = h0_r[...]
    ys = (y0_r, y1_r, y2_r)
    for j in range(N_HOP):
        dinv = lax.rsqrt(1.0 + d[j] + d[N_HOP + j])
        agg = aggp_r[j * NC] + aggp_r[j * NC + 1] + ys[j][...]
        out = agg * dinv[:, None] + bias_r[j][None, :]
        h = h + jnp.maximum(out, 0.0)
    m = jnp.max(h, axis=1, keepdims=True)
    e = jnp.exp(h - m)
    lse = jnp.log(jnp.sum(e, axis=1, keepdims=True))
    out_r[...] = h - m - lse


def _fin_call(h0, y0, y1, y2, aggp, bias_h, degp):
    oblk = pl.BlockSpec((RB, D_OUT), lambda i: (i, 0))
    ablk = pl.BlockSpec((N_HOP * NC, RB, WPAD), lambda i: (0, i, 0))
    dblk = pl.BlockSpec((NC * N_HOP, RB), lambda i: (0, i))
    return pl.pallas_call(
        _fin_body,
        grid=(NRB,),
        in_specs=[oblk, oblk, oblk, oblk, ablk,
                  pl.BlockSpec(bias_h.shape, lambda i: (0, 0)),
                  dblk],
        out_specs=oblk,
        out_shape=jax.ShapeDtypeStruct((N, D_OUT), jnp.float32),
    )(h0, y0, y1, y2, aggp, bias_h, degp)


def kernel(bb0, bb1, bb2, bb3, edge0, edge1, edge2, comb_w, ego_W, ego_b,
           W0, b0, W1, b1, W2, b2):
    del bb0
    alphas = jax.nn.softmax(comb_w, axis=1)
    wall = jnp.stack([ego_W, W0, W1, W2], axis=0)
    t = alphas[:, :, None, None] * wall[:, None, :, :]
    wstack = t.transpose(1, 2, 0, 3).reshape(3, D_IN, (N_HOP + 1) * D_OUT)
    biascat = jnp.concatenate(
        [ego_b, jnp.zeros((N_HOP * D_OUT,), jnp.float32)]).reshape(1, -1)
    bias_h = jnp.stack([b0, b1, b2], axis=0)

    degp = _deg_kernel(edge0, edge1, edge2).reshape(NC * N_HOP, NPAD)
    h0, y0, y1, y2 = _mm_call(bb1, bb2, bb3, wstack, biascat, degp)
    aggp = _agg_kernel(y0, y1, y2, edge0, edge1,
                       edge2).reshape(N_HOP * NC, NPAD, WPAD)
    return _fin_call(h0, y0, y1, y2, aggp, bias_h, degp)
